# baseline (speedup 1.0000x reference)
import functools

import jax
import jax.numpy as jnp
from jax import lax
from jax.experimental import pallas as pl
from jax.experimental.pallas import tpu as pltpu
from jax.experimental.pallas import tpu_sc as plsc

N_TOKENS = 32768
B = 16
D = 1024
L = 16
NC = 2
NS = 16
NW = NC * NS

SC_ROWS = 17920
ROWS_PER_W = SC_ROWS // NW
K = 6
LEAD = 3
BLK = 16
NBLK = ROWS_PER_W // BLK

TC_BLK = 256
TC_ROWS = N_TOKENS - SC_ROWS


def _norm_sc_body(img_hbm, par_hbm, out_hbm,
                  par_v, bm_v, bi_v, bufs, sems_in, sems_out):
    wid = lax.axis_index("s") * NC + lax.axis_index("c")
    base = wid * ROWS_PER_W

    pltpu.sync_copy(par_hbm, par_v)
    cu = par_v[pl.ds(0, L)]
    si = par_v[pl.ds(L, L)]
    mp = plsc.bitcast(par_v[pl.ds(2 * L, L)], jnp.float32)
    sp = plsc.bitcast(par_v[pl.ds(3 * L, L)], jnp.float32)
    bm_v[...] = mp
    bi_v[...] = sp
    bm_v[...] = plsc.load_gather(bm_v, [si])
    bi_v[...] = 1.0 / plsc.load_gather(bi_v, [si])

    def in_cp(blk, slot):
        row0 = base + blk * BLK
        return pltpu.make_async_copy(
            img_hbm.at[pl.ds(row0, BLK)], bufs.at[slot], sems_in.at[slot])

    def out_cp(blk, slot):
        row0 = base + blk * BLK
        return pltpu.make_async_copy(
            bufs.at[slot], out_hbm.at[pl.ds(row0, BLK)], sems_out.at[slot])

    def compute(blk, slot):
        buf = bufs.at[slot]
        row0 = base + blk * BLK

        def row_body(i, c2):
            r = row0 + i
            seg = plsc.all_reduce_population_count(
                cu <= jnp.full((L,), r, jnp.int32))
            m = plsc.load_gather(bm_v, [seg])
            iv = plsc.load_gather(bi_v, [seg])
            for c in range(D // L):
                x = buf[i, pl.ds(c * L, L)]
                buf[i, pl.ds(c * L, L)] = (x - m) * iv
            return c2

        lax.fori_loop(0, BLK, row_body, 0, unroll=False)

    for b in range(LEAD):
        in_cp(b, b).start()

    def step(blk, carry):
        slot = lax.rem(blk, K)
        slot_next = lax.rem(blk + LEAD, K)
        @pl.when(blk + LEAD >= K)
        def _():
            out_cp(blk + LEAD - K, slot_next).wait()

        @pl.when(blk + LEAD < NBLK)
        def _():
            in_cp(blk + LEAD, slot_next).start()

        in_cp(blk, slot).wait()
        compute(blk, slot)
        out_cp(blk, slot).start()
        return carry

    lax.fori_loop(0, NBLK, step, 0, unroll=False)
    for b in range(NBLK - (K - LEAD), NBLK):
        out_cp(b, b % K).wait()


_norm_sc = functools.partial(
    pl.kernel,
    out_type=jax.ShapeDtypeStruct((SC_ROWS, D), jnp.float32),
    mesh=plsc.VectorSubcoreMesh(core_axis_name="c", subcore_axis_name="s"),
    compiler_params=pltpu.CompilerParams(needs_layout_passes=False),
    scratch_types=[
        pltpu.VMEM((4 * L,), jnp.int32),
        pltpu.VMEM((L,), jnp.float32),
        pltpu.VMEM((L,), jnp.float32),
        pltpu.VMEM((K, BLK, D), jnp.float32),
        pltpu.SemaphoreType.DMA((K,)),
        pltpu.SemaphoreType.DMA((K,)),
    ],
)(_norm_sc_body)


def _norm_tc_body(cu_ref, si_ref, mp_ref, sp_ref, img_ref, out_ref):
    i = pl.program_id(0)
    r = (SC_ROWS + i * TC_BLK
         + lax.broadcasted_iota(jnp.int32, (TC_BLK, 1), 0))
    seg = jnp.zeros((TC_BLK, 1), jnp.int32)
    for k in range(B):
        seg += (cu_ref[k] <= r).astype(jnp.int32)
    mean = jnp.zeros((TC_BLK, 1), jnp.float32)
    inv = jnp.ones((TC_BLK, 1), jnp.float32)
    for b in range(B):
        sel = seg == b
        mean = jnp.where(sel, mp_ref[si_ref[b]], mean)
        inv = jnp.where(sel, 1.0 / sp_ref[si_ref[b]], inv)
    out_ref[...] = (img_ref[...] - mean) * inv


_norm_tc = pl.pallas_call(
    _norm_tc_body,
    grid=(TC_ROWS // TC_BLK,),
    in_specs=[
        pl.BlockSpec(memory_space=pltpu.SMEM),
        pl.BlockSpec(memory_space=pltpu.SMEM),
        pl.BlockSpec(memory_space=pltpu.SMEM),
        pl.BlockSpec(memory_space=pltpu.SMEM),
        pl.BlockSpec((TC_BLK, D), lambda i: (i + SC_ROWS // TC_BLK, 0)),
    ],
    out_specs=pl.BlockSpec((TC_BLK, D), lambda i: (i, 0)),
    out_shape=jax.ShapeDtypeStruct((TC_ROWS, D), jnp.float32),
)


@jax.jit
def kernel(img, stat_idx, cu_seqlens, means, stds):
    nstats = means.shape[0]
    mp = jnp.concatenate(
        [means.astype(jnp.float32),
         jnp.zeros((L - nstats,), jnp.float32)])
    sp = jnp.concatenate(
        [stds.astype(jnp.float32),
         jnp.ones((L - nstats,), jnp.float32)])
    cu_tail = cu_seqlens[1:].astype(jnp.int32)
    sidx = stat_idx.astype(jnp.int32)
    par = jnp.concatenate([
        cu_tail,
        sidx,
        lax.bitcast_convert_type(mp, jnp.int32),
        lax.bitcast_convert_type(sp, jnp.int32),
    ])
    out_sc = _norm_sc(img, par)
    out_tc = _norm_tc(cu_tail, sidx, mp, sp, img)
    return jnp.concatenate([out_sc, out_tc], axis=0)


# speedup vs baseline: 1.2056x; 1.2056x over previous
import functools

import jax
import jax.numpy as jnp
from jax import lax
from jax.experimental import pallas as pl
from jax.experimental.pallas import tpu as pltpu
from jax.experimental.pallas import tpu_sc as plsc

N_TOKENS = 32768
B = 16
D = 1024
L = 16
NC = 2
NS = 16
NW = NC * NS
ROWS_PER_W = N_TOKENS // NW
NGRP = ROWS_PER_W // L

TC_BLK = 256


def _param_sc_body(par_hbm, ab_hbm, par_v, bm_v, bi_v, a16_v, b16_v, buf):
    wid = lax.axis_index("s") * NC + lax.axis_index("c")
    base = wid * ROWS_PER_W

    pltpu.sync_copy(par_hbm, par_v)
    cu_v = par_v.at[pl.ds(0, L)]
    si = par_v[pl.ds(L, L)]
    mp = plsc.bitcast(par_v[pl.ds(2 * L, L)], jnp.float32)
    sp = plsc.bitcast(par_v[pl.ds(3 * L, L)], jnp.float32)
    bm_v[...] = mp
    bi_v[...] = sp
    bm_v[...] = plsc.load_gather(bm_v, [si])
    bi_v[...] = 1.0 / plsc.load_gather(bi_v, [si])

    lanes = lax.iota(jnp.int32, L)
    lane0 = lanes == 0
    zero_rt = jnp.minimum(si, 0)
    cu_splat = [plsc.load_gather(cu_v, [zero_rt + k]) for k in range(B)]

    HALF = ROWS_PER_W // 2

    for h in range(2):
        def group(g, carry):
            rvec = base + h * HALF + g * L + lanes
            seg = jnp.zeros((L,), jnp.int32)
            for k in range(B):
                seg += (cu_splat[k] <= rvec).astype(jnp.int32)
            a16_v[...] = plsc.load_gather(bi_v, [seg])
            b16_v[...] = (-plsc.load_gather(bm_v, [seg])
                          * a16_v[...])
            for j in range(L):
                jidx = jnp.full((L,), j, jnp.int32)
                a = plsc.load_gather(a16_v, [jidx])
                bb = plsc.load_gather(b16_v, [jidx])
                buf[g * L + j, pl.ds(0, L)] = jnp.where(lane0, a, bb)
            return carry

        lax.fori_loop(0, HALF // L, group, 0, unroll=False)
        pltpu.sync_copy(buf, ab_hbm.at[pl.ds(base + h * HALF, HALF)])


_param_sc = functools.partial(
    pl.kernel,
    out_type=jax.ShapeDtypeStruct((N_TOKENS, 128), jnp.float32),
    mesh=plsc.VectorSubcoreMesh(core_axis_name="c", subcore_axis_name="s"),
    compiler_params=pltpu.CompilerParams(needs_layout_passes=False),
    scratch_types=[
        pltpu.VMEM((4 * L,), jnp.int32),
        pltpu.VMEM((L,), jnp.float32),
        pltpu.VMEM((L,), jnp.float32),
        pltpu.VMEM((L,), jnp.float32),
        pltpu.VMEM((L,), jnp.float32),
        pltpu.VMEM((ROWS_PER_W // 2, 128), jnp.float32),
    ],
)(_param_sc_body)


def _norm_tc_body(ab_ref, img_ref, out_ref):
    a = ab_ref[:, 0:1]
    bb = ab_ref[:, 1:2]
    out_ref[...] = img_ref[...] * a + bb


_norm_tc = pl.pallas_call(
    _norm_tc_body,
    grid=(N_TOKENS // TC_BLK,),
    in_specs=[
        pl.BlockSpec((TC_BLK, 128), lambda i: (i, 0)),
        pl.BlockSpec((TC_BLK, D), lambda i: (i, 0)),
    ],
    out_specs=pl.BlockSpec((TC_BLK, D), lambda i: (i, 0)),
    out_shape=jax.ShapeDtypeStruct((N_TOKENS, D), jnp.float32),
)


@jax.jit
def kernel(img, stat_idx, cu_seqlens, means, stds):
    nstats = means.shape[0]
    mp = jnp.concatenate(
        [means.astype(jnp.float32),
         jnp.zeros((L - nstats,), jnp.float32)])
    sp = jnp.concatenate(
        [stds.astype(jnp.float32),
         jnp.ones((L - nstats,), jnp.float32)])
    par = jnp.concatenate([
        cu_seqlens[1:].astype(jnp.int32),
        stat_idx.astype(jnp.int32),
        lax.bitcast_convert_type(mp, jnp.int32),
        lax.bitcast_convert_type(sp, jnp.int32),
    ])
    ab = _param_sc(par)
    return _norm_tc(ab, img)


# speedup vs baseline: 1.7685x; 1.4668x over previous
import functools

import jax
import jax.numpy as jnp
from jax import lax
from jax.experimental import pallas as pl
from jax.experimental.pallas import tpu as pltpu
from jax.experimental.pallas import tpu_sc as plsc

N_TOKENS = 32768
B = 16
D = 1024
L = 16
NC = 2
NS = 16
NW = NC * NS
ROWS_PER_W = N_TOKENS // NW
K = 6
LEAD = 3
BLK = 16
NBLK = ROWS_PER_W // BLK


def _norm_body(img_hbm, par_hbm, out_hbm,
               par_v, bm_v, bi_v, bufs, sems_in, sems_out):
    wid = lax.axis_index("s") * NC + lax.axis_index("c")
    base = wid * ROWS_PER_W

    pltpu.sync_copy(par_hbm, par_v)
    cu = par_v[pl.ds(0, L)]
    si = par_v[pl.ds(L, L)]
    mp = plsc.bitcast(par_v[pl.ds(2 * L, L)], jnp.float32)
    sp = plsc.bitcast(par_v[pl.ds(3 * L, L)], jnp.float32)
    bm_v[...] = mp
    bi_v[...] = sp
    bm_v[...] = plsc.load_gather(bm_v, [si])
    bi_v[...] = 1.0 / plsc.load_gather(bi_v, [si])

    def in_cp(blk, slot):
        row0 = base + blk * BLK
        return pltpu.make_async_copy(
            img_hbm.at[pl.ds(row0, BLK)], bufs.at[slot], sems_in.at[slot])

    def out_cp(blk, slot):
        row0 = base + blk * BLK
        return pltpu.make_async_copy(
            bufs.at[slot], out_hbm.at[pl.ds(row0, BLK)], sems_out.at[slot])

    def compute(blk, slot):
        buf = bufs.at[slot]
        row0 = base + blk * BLK

        def row_body(i, c2):
            r = row0 + i
            seg = plsc.all_reduce_population_count(
                cu <= jnp.full((L,), r, jnp.int32))
            m = plsc.load_gather(bm_v, [seg])
            iv = plsc.load_gather(bi_v, [seg])
            for c in range(D // L):
                x = buf[i, pl.ds(c * L, L)]
                buf[i, pl.ds(c * L, L)] = (x - m) * iv
            return c2

        lax.fori_loop(0, BLK, row_body, 0, unroll=False)

    for b in range(LEAD):
        in_cp(b, b).start()

    def step(blk, carry):
        slot = lax.rem(blk, K)
        slot_next = lax.rem(blk + LEAD, K)
        @pl.when(blk + LEAD >= K)
        def _():
            out_cp(blk + LEAD - K, slot_next).wait()

        @pl.when(blk + LEAD < NBLK)
        def _():
            in_cp(blk + LEAD, slot_next).start()

        in_cp(blk, slot).wait()
        compute(blk, slot)
        out_cp(blk, slot).start()
        return carry

    lax.fori_loop(0, NBLK, step, 0, unroll=False)
    for b in range(NBLK - (K - LEAD), NBLK):
        out_cp(b, b % K).wait()


_norm_sc = functools.partial(
    pl.kernel,
    out_type=jax.ShapeDtypeStruct((N_TOKENS, D), jnp.float32),
    mesh=plsc.VectorSubcoreMesh(core_axis_name="c", subcore_axis_name="s"),
    compiler_params=pltpu.CompilerParams(needs_layout_passes=False),
    scratch_types=[
        pltpu.VMEM((4 * L,), jnp.int32),
        pltpu.VMEM((L,), jnp.float32),
        pltpu.VMEM((L,), jnp.float32),
        pltpu.VMEM((K, BLK, D), jnp.float32),
        pltpu.SemaphoreType.DMA((K,)),
        pltpu.SemaphoreType.DMA((K,)),
    ],
)(_norm_body)


@jax.jit
def kernel(img, stat_idx, cu_seqlens, means, stds):
    nstats = means.shape[0]
    mp = jnp.concatenate(
        [means.astype(jnp.float32),
         jnp.zeros((L - nstats,), jnp.float32)])
    sp = jnp.concatenate(
        [stds.astype(jnp.float32),
         jnp.ones((L - nstats,), jnp.float32)])
    par = jnp.concatenate([
        cu_seqlens[1:].astype(jnp.int32),
        stat_idx.astype(jnp.int32),
        lax.bitcast_convert_type(mp, jnp.int32),
        lax.bitcast_convert_type(sp, jnp.int32),
    ])
    return _norm_sc(img, par)
